# baseline (speedup 1.0000x reference)
import jax
import jax.numpy as jnp
from itertools import combinations
from jax.experimental import pallas as pl

_N_WIRES = 16
_N_SHOTS = 1024.0
_IDX = [(1 << (_N_WIRES - 1 - a)) + (1 << (_N_WIRES - 1 - b))
        for a, b in combinations(range(_N_WIRES), 2)]
_NB = len(_IDX)


def _body(x_ref, o_ref):
    x = x_ref[...]
    s = jnp.sum(x, axis=1)
    scale = _N_SHOTS / s
    cols = [x[:, c] for c in _IDX]
    g = jnp.stack(cols, axis=1)
    o_ref[...] = g * scale[:, None]


def kernel(state, interpret=False):
    B, N = state.shape
    BR = 64
    return pl.pallas_call(
        _body,
        grid=(B // BR,),
        in_specs=[pl.BlockSpec((BR, N), lambda i: (i, 0))],
        out_specs=pl.BlockSpec((BR, _NB), lambda i: (i, 0)),
        out_shape=jax.ShapeDtypeStruct((B, _NB), jnp.float32),
        interpret=interpret,
    )(state)


# speedup vs baseline: 1.0282x; 1.0282x over previous
import jax
import jax.numpy as jnp
from itertools import combinations
from jax.experimental import pallas as pl

_N_WIRES = 16
_N_SHOTS = 1024.0
_IDX = [(1 << (_N_WIRES - 1 - a)) + (1 << (_N_WIRES - 1 - b))
        for a, b in combinations(range(_N_WIRES), 2)]
_NB = len(_IDX)


def _body2(x1_ref, x2_ref, o_ref):
    x1 = x1_ref[...]
    x2 = x2_ref[...]
    s = jnp.sum(x1, axis=1) + jnp.sum(x2, axis=1)
    scale = _N_SHOTS / s
    half = 1 << (_N_WIRES - 1)
    cols = [x1[:, c] if c < half else x2[:, c - half] for c in _IDX]
    g = jnp.stack(cols, axis=1)
    o_ref[...] = g * scale[:, None]


def kernel(state, interpret=False):
    B, N = state.shape
    BR = 32
    H = N // 2
    return pl.pallas_call(
        _body2,
        grid=(B // BR,),
        in_specs=[pl.BlockSpec((BR, H), lambda i: (i, 0)),
                  pl.BlockSpec((BR, H), lambda i: (i, 1))],
        out_specs=pl.BlockSpec((BR, _NB), lambda i: (i, 0)),
        out_shape=jax.ShapeDtypeStruct((B, _NB), jnp.float32),
        interpret=interpret,
    )(state, state)
